# Initial kernel scaffold; baseline (speedup 1.0000x reference)
import jax
import jax.numpy as jnp
from jax.experimental import pallas as pl


def kernel(x, ln_pre_w, router_w, sh_gate, sh_up, sh_down, eg, eu, ed, layer_scale, ln_post_w):
    raise NotImplementedError("write your pallas kernel here")


# speedup vs baseline: 1.5918x; 1.5918x over previous
import functools

import jax
import jax.numpy as jnp
from jax.experimental import pallas as pl
from jax.experimental.pallas import tpu as pltpu

EPS = 1e-6


def _moe_kernel(xp_ref, lnpre_ref, rw_ref, sg_ref, su_ref, sd_ref,
                eg_ref, eu_ref, ed_ref, ls_ref, lnpost_ref,
                out_ref, fn_ref, i1_ref, i2_ref, w1_ref, acc_ref, *, n_experts):
    e = pl.program_id(0)

    @pl.when(e == 0)
    def _prologue():
        h = xp_ref[...]
        var = jnp.mean(h * h, axis=-1, keepdims=True)
        fn = (h * jax.lax.rsqrt(var + EPS)) * lnpre_ref[...]
        fn_ref[...] = fn
        g = jnp.dot(fn, sg_ref[...], preferred_element_type=jnp.float32)
        u = jnp.dot(fn, su_ref[...], preferred_element_type=jnp.float32)
        acc_ref[...] = jnp.dot(jax.nn.silu(g) * u, sd_ref[...],
                               preferred_element_type=jnp.float32)
        logits = jnp.dot(fn, rw_ref[...], preferred_element_type=jnp.float32)
        n, ne = logits.shape
        iota = jax.lax.broadcasted_iota(jnp.int32, (n, ne), 1)
        m1 = jnp.max(logits, axis=-1, keepdims=True)
        i1 = jnp.min(jnp.where(logits == m1, iota, ne), axis=-1, keepdims=True)
        masked = jnp.where(iota == i1, -jnp.inf, logits)
        m2 = jnp.max(masked, axis=-1, keepdims=True)
        i2 = jnp.min(jnp.where(masked == m2, iota, ne), axis=-1, keepdims=True)
        i1_ref[...] = i1
        i2_ref[...] = i2
        w1_ref[...] = jax.nn.sigmoid(m1 - m2)

    fn = fn_ref[...]
    g = jnp.dot(fn, eg_ref[0], preferred_element_type=jnp.float32)
    u = jnp.dot(fn, eu_ref[0], preferred_element_type=jnp.float32)
    hmid = jax.nn.silu(g) * u
    w1 = w1_ref[...]
    ce = (jnp.where(i1_ref[...] == e, w1, 0.0)
          + jnp.where(i2_ref[...] == e, 1.0 - w1, 0.0))
    acc_ref[...] += jnp.dot(hmid * ce, ed_ref[0],
                            preferred_element_type=jnp.float32)

    @pl.when(e == n_experts - 1)
    def _epilogue():
        a = acc_ref[...] * ls_ref[...]
        var = jnp.mean(a * a, axis=-1, keepdims=True)
        out_ref[...] = (a * jax.lax.rsqrt(var + EPS)) * lnpost_ref[...]


@jax.jit
def kernel(x, ln_pre_w, router_w, sh_gate, sh_up, sh_down, eg, eu, ed,
           layer_scale, ln_post_w):
    b, t, d = x.shape
    in_dim = ln_pre_w.shape[0]
    k_pool = in_dim // d
    t2 = (t // k_pool) * k_pool
    n = b * (t2 // k_pool)
    n_experts = router_w.shape[1]
    hid = sh_gate.shape[1]
    out_dim = sh_down.shape[1]

    xp = x[:, :t2, :].reshape(n, in_dim)
    full = lambda shape: pl.BlockSpec(shape, lambda e: (0,) * len(shape))

    out = pl.pallas_call(
        functools.partial(_moe_kernel, n_experts=n_experts),
        grid=(n_experts,),
        in_specs=[
            full((n, in_dim)),
            full((1, in_dim)),
            full((in_dim, n_experts)),
            full((in_dim, hid)),
            full((in_dim, hid)),
            full((hid, out_dim)),
            pl.BlockSpec((1, in_dim, hid), lambda e: (e, 0, 0)),
            pl.BlockSpec((1, in_dim, hid), lambda e: (e, 0, 0)),
            pl.BlockSpec((1, hid, out_dim), lambda e: (e, 0, 0)),
            full((1, out_dim)),
            full((1, out_dim)),
        ],
        out_specs=full((n, out_dim)),
        out_shape=jax.ShapeDtypeStruct((n, out_dim), jnp.float32),
        scratch_shapes=[
            pltpu.VMEM((n, in_dim), jnp.float32),
            pltpu.VMEM((n, 1), jnp.int32),
            pltpu.VMEM((n, 1), jnp.int32),
            pltpu.VMEM((n, 1), jnp.float32),
            pltpu.VMEM((n, out_dim), jnp.float32),
        ],
    )(xp, ln_pre_w.reshape(1, in_dim), router_w, sh_gate, sh_up, sh_down,
      eg, eu, ed, layer_scale.reshape(1, out_dim), ln_post_w.reshape(1, out_dim))
    return out.reshape(b, t2 // k_pool, out_dim)
